# baseline (speedup 1.0000x reference)
import functools

import jax
import jax.numpy as jnp
from jax import lax
from jax.experimental import pallas as pl
from jax.experimental.pallas import tpu as pltpu
from jax.experimental.pallas import tpu_sc as plsc

_CH = 128
_NSEM = 4


@functools.lru_cache(maxsize=None)
def _build(batch, emb, n_users, n_items):
    info = plsc.get_sparse_core_info()
    nw = info.num_cores * info.num_subcores
    lanes = info.num_lanes
    b_per_w = batch // nw
    assert b_per_w * nw == batch and b_per_w % _CH == 0 and _CH % lanes == 0
    nch = b_per_w // _CH

    mesh = plsc.VectorSubcoreMesh(core_axis_name="c", subcore_axis_name="s")

    @functools.partial(
        pl.kernel,
        mesh=mesh,
        compiler_params=pltpu.CompilerParams(use_tc_tiling_on_sc=True),
        out_type=(
            jax.ShapeDtypeStruct((batch, emb), jnp.float32),
            jax.ShapeDtypeStruct((batch, emb), jnp.float32),
        ),
        scratch_types=[
            pltpu.VMEM((b_per_w,), jnp.int32),
            pltpu.VMEM((b_per_w,), jnp.int32),
            pltpu.VMEM((_CH, emb), jnp.float32),
            pltpu.VMEM((_CH, emb), jnp.float32),
            pltpu.VMEM_SHARED((info.num_subcores, _CH, emb), jnp.float32),
            [pltpu.SemaphoreType.DMA] * _NSEM,
            [pltpu.SemaphoreType.DMA] * _NSEM,
            pltpu.SemaphoreType.DMA,
        ],
    )
    def k(user_hbm, item_hbm, ut_hbm, it_hbm, uout, iout,
          uidx, iidx, urows, irows, ispm, usems, isems, wsem):
        sid = lax.axis_index("s")
        wid = sid * info.num_cores + lax.axis_index("c")
        base = wid * b_per_w
        pltpu.sync_copy(user_hbm.at[pl.ds(base, b_per_w)], uidx)
        pltpu.sync_copy(item_hbm.at[pl.ds(base, b_per_w)], iidx)

        def chunk(c, carry):
            def rbody(j, carry2):
                b = j * lanes
                uv = uidx[pl.ds(c * _CH + b, lanes)]
                iv = iidx[pl.ds(c * _CH + b, lanes)]
                for kk in range(lanes):
                    pltpu.async_copy(ut_hbm.at[uv[kk]],
                                     urows.at[b + kk], usems[kk % _NSEM])
                    pltpu.async_copy(it_hbm.at[iv[kk]],
                                     ispm.at[sid, b + kk], isems[kk % _NSEM])
                return carry2

            lax.fori_loop(0, _CH // lanes, rbody, 0)

            def rdrain(j, carry2):
                for s in range(_NSEM):
                    pltpu.make_async_copy(ut_hbm.at[0], urows.at[0],
                                          usems[s]).wait()
                    pltpu.make_async_copy(it_hbm.at[0],
                                          ispm.at[0, 0], isems[s]).wait()
                return carry2

            lax.fori_loop(0, _CH // _NSEM, rdrain, 0)

            pltpu.sync_copy(ispm.at[sid], irows)

            cbase = base + c * _CH
            ucp = pltpu.make_async_copy(
                urows, uout.at[pl.ds(cbase, _CH)], wsem)
            icp = pltpu.make_async_copy(
                irows, iout.at[pl.ds(cbase, _CH)], wsem)
            ucp.start()
            icp.start()
            ucp.wait()
            icp.wait()
            return carry

        lax.fori_loop(0, nch, chunk, 0)

    return k


def kernel(user, item, user_table, item_table):
    batch, = user.shape
    n_users, emb = user_table.shape
    n_items, _ = item_table.shape
    k = _build(batch, emb, n_users, n_items)
    return k(user, item, user_table, item_table)
